# baseline (speedup 1.0000x reference)
import functools

import jax
import jax.numpy as jnp
import numpy as np
from jax import lax
from jax.experimental import pallas as pl
from jax.experimental.pallas import tpu as pltpu
from jax.experimental.pallas import tpu_sc as plsc

N_NODES = 10000
N_HEDGES = 5000
N_INC = 320000
FEAT = 128
DIM = 128
HEADS = 4
HID = DIM // HEADS
NCLS = 40
NLAYERS = 2

WID = DIM + 16
NW = 32
CHUNK = 80
ZROWS = 640

_S_np = np.zeros((DIM, HEADS), np.float32)
for _h in range(HEADS):
    _S_np[_h * HID:(_h + 1) * HID, _h] = 1.0


def _ln(o, g, b):
    m = jnp.mean(o, axis=-1, keepdims=True)
    c = o - m
    v = jnp.mean(c * c, axis=-1, keepdims=True)
    return c * jax.lax.rsqrt(v + 1e-5) * g + b


def _elu(o):
    return jnp.where(o > 0, o, jnp.exp(jnp.minimum(o, 0.0)) - 1.0)


def _pre_block(x, Wk, Wv, att, S, ST):
    Kx = jnp.dot(x, Wk, preferred_element_type=jnp.float32)
    alpha = jnp.dot(Kx * att, S, preferred_element_type=jnp.float32)
    gmax = jnp.max(alpha, axis=0, keepdims=True)
    ea = jnp.exp(alpha - gmax)
    Vx = jnp.dot(x, Wv, preferred_element_type=jnp.float32)
    eaexp = jnp.dot(ea, ST, preferred_element_type=jnp.float32)
    pad = jnp.zeros((x.shape[0], WID - DIM - HEADS), jnp.float32)
    return jnp.concatenate([Vx * eaexp, ea, pad], axis=1)


def _post_block(raw0, raw1, att, ST, ln0_g, ln0_b, Wff, bff, ln1_g, ln1_b):
    raw = raw0 + raw1
    s = jnp.dot(raw[:, DIM:DIM + HEADS], ST,
                preferred_element_type=jnp.float32) + 1e-16
    o = raw[:, :DIM] / s + att
    o = _ln(o, ln0_g, ln0_b)
    o = o + jax.nn.relu(jnp.dot(o, Wff, preferred_element_type=jnp.float32) + bff)
    o = _ln(o, ln1_g, ln1_b)
    return _elu(o)


def _t0_body(x_ref, bng, bnb, bnm, bnv, Wk, Wv, att, S, ST, u_ref):
    x = (x_ref[...] - bnm[...]) * jax.lax.rsqrt(bnv[...] + 1e-5) * bng[...] + bnb[...]
    u_ref[...] = _pre_block(x, Wk[...], Wv[...], att[...], S[...], ST[...])


def _tmid_body(r0, r1, att_a, g0, b0, Wff, bff, g1, b1,
               Wk, Wv, att_b, S, ST, u_ref, x_ref):
    x = _post_block(r0[...], r1[...], att_a[...], ST[...], g0[...], b0[...],
                    Wff[...], bff[...], g1[...], b1[...])
    x_ref[...] = x
    u_ref[...] = _pre_block(x, Wk[...], Wv[...], att_b[...], S[...], ST[...])


def _t4_body(r0, r1, att_a, g0, b0, Wff, bff, g1, b1, ST,
             x1_ref, Wc1, Wc2, bc, out_ref):
    x2 = _post_block(r0[...], r1[...], att_a[...], ST[...], g0[...], b0[...],
                     Wff[...], bff[...], g1[...], b1[...])
    logits = (jnp.dot(x1_ref[...], Wc1[...], preferred_element_type=jnp.float32)
              + jnp.dot(x2, Wc2[...], preferred_element_type=jnp.float32)
              + bc[...])
    z = logits - jnp.max(logits, axis=-1, keepdims=True)
    out_ref[...] = z - jnp.log(jnp.sum(jnp.exp(z), axis=-1, keepdims=True))


def _tc(body, out_shape, *args):
    return pl.pallas_call(body, out_shape=out_shape)(*args)


@functools.lru_cache(maxsize=None)
def _make_sc_segsum(nseg, n_rows):
    chunk = 128 if nseg <= 5000 else 64
    nseg_pad = ((nseg + 127) // 128) * 128
    rpt = nseg_pad // 16
    nsup = -(-N_INC // (NW * chunk))
    nsup += nsup % 2
    crows = nsup
    inc_pad = NW * crows * chunk

    mesh = plsc.VectorSubcoreMesh(core_axis_name="c", subcore_axis_name="s")

    @functools.partial(
        pl.kernel,
        mesh=mesh,
        compiler_params=pltpu.CompilerParams(use_tc_tiling_on_sc=False),
        out_type=jax.ShapeDtypeStruct((2 * nseg_pad, WID), jnp.float32),
        scratch_types=[
            pltpu.VMEM((crows, chunk), jnp.int32),
            pltpu.VMEM((crows, chunk), jnp.int32),
            [pltpu.VMEM((chunk, WID), jnp.float32)] * 2,
            pltpu.VMEM_SHARED((nseg_pad, WID), jnp.float32),
            [pltpu.SemaphoreType.DMA] * 2,
        ],
    )
    def segsum(u_hbm, src_hbm, dst_hbm, zeros_hbm, out_hbm,
               src_v, dst_v, bufs, acc, sems):
        c = lax.axis_index("c")
        s = lax.axis_index("s")
        w = (1 - c) * 16 + s

        pltpu.sync_copy(src_hbm.at[pl.ds(w * crows, crows)], src_v)
        pltpu.sync_copy(dst_hbm.at[pl.ds(w * crows, crows)], dst_v)
        pltpu.sync_copy(zeros_hbm.at[pl.ds(0, rpt)], acc.at[pl.ds(s * rpt, rpt)])
        plsc.subcore_barrier()

        def fire_g(sup, bk):
            pltpu.async_copy(u_hbm.at[src_v.at[sup]], bufs[bk], sems[bk])

        def drain_g(sup, bk):
            pltpu.make_async_copy(u_hbm.at[src_v.at[sup]], bufs[bk],
                                  sems[bk]).wait()

        def scat(sup, bk):
            pltpu.sync_copy(bufs[bk], acc.at[dst_v.at[sup]], add=True)

        fire_g(0, 0)

        def body(t, carry):
            s1 = 2 * t - 1
            fire_g(s1, 1)
            drain_g(s1 - 1, 0)
            scat(s1 - 1, 0)
            fire_g(s1 + 1, 0)
            drain_g(s1, 1)
            scat(s1, 1)
            return carry

        lax.fori_loop(1, nsup // 2, body, 0)

        fire_g(nsup - 1, 1)
        drain_g(nsup - 2, 0)
        scat(nsup - 2, 0)
        drain_g(nsup - 1, 1)
        scat(nsup - 1, 1)
        plsc.subcore_barrier()

        row0 = c * nseg_pad + s * rpt
        pltpu.sync_copy(acc.at[pl.ds(s * rpt, rpt)], out_hbm.at[pl.ds(row0, rpt)])

    return segsum, nseg_pad, inc_pad, chunk


def _sc_segsum(U, src, dst, nseg, zeros):
    fn, nseg_pad, inc_pad, chunk = _make_sc_segsum(nseg, U.shape[0])
    pad = inc_pad - N_INC
    dump = nseg + jnp.arange(pad, dtype=dst.dtype) % (nseg_pad - nseg)
    srcr = jnp.concatenate([src, jnp.zeros((pad,), src.dtype)]).reshape(-1, chunk)
    dstr = jnp.concatenate([dst, dump]).reshape(-1, chunk)
    out = fn(U, srcr, dstr, zeros)
    return out[:nseg], out[nseg_pad:nseg_pad + nseg]


def kernel(node_x, n2e_nodes_map, n2e_batch, e2n_edges_map, e2n_batch, params):
    S = jnp.asarray(_S_np)
    ST = jnp.asarray(_S_np.T)
    zeros = jnp.zeros((ZROWS, WID), jnp.float32)

    def row(v):
        return jnp.reshape(v, (1, -1)).astype(jnp.float32)

    def pre_args(p):
        return (p['Wk'], p['Wv'], row(p['att_r']), S, ST)

    def post_args(p):
        return (row(p['att_r']), row(p['ln0_g']), row(p['ln0_b']),
                p['Wff'], row(p['bff']), row(p['ln1_g']), row(p['ln1_b']))

    n2e0, n2e1 = params['n2e']
    e2n0, e2n1 = params['e2n']

    U_a = _tc(_t0_body, jax.ShapeDtypeStruct((N_NODES, WID), jnp.float32),
              node_x, row(params['bn_g']), row(params['bn_b']),
              row(params['bn_m']), row(params['bn_v']), *pre_args(n2e0))
    ra0, ra1 = _sc_segsum(U_a, n2e_nodes_map, n2e_batch, N_HEDGES, zeros)

    U_b, _ = _tc(_tmid_body,
                 (jax.ShapeDtypeStruct((N_HEDGES, WID), jnp.float32),
                  jax.ShapeDtypeStruct((N_HEDGES, DIM), jnp.float32)),
                 ra0, ra1, *post_args(n2e0), *pre_args(e2n0))
    rb0, rb1 = _sc_segsum(U_b, e2n_edges_map, e2n_batch, N_NODES, zeros)

    U_c, x1 = _tc(_tmid_body,
                  (jax.ShapeDtypeStruct((N_NODES, WID), jnp.float32),
                   jax.ShapeDtypeStruct((N_NODES, DIM), jnp.float32)),
                  rb0, rb1, *post_args(e2n0), *pre_args(n2e1))
    rc0, rc1 = _sc_segsum(U_c, n2e_nodes_map, n2e_batch, N_HEDGES, zeros)

    U_d, _ = _tc(_tmid_body,
                 (jax.ShapeDtypeStruct((N_HEDGES, WID), jnp.float32),
                  jax.ShapeDtypeStruct((N_HEDGES, DIM), jnp.float32)),
                 rc0, rc1, *post_args(n2e1), *pre_args(e2n1))
    rd0, rd1 = _sc_segsum(U_d, e2n_edges_map, e2n_batch, N_NODES, zeros)

    out = _tc(_t4_body, jax.ShapeDtypeStruct((N_NODES, NCLS), jnp.float32),
              rd0, rd1, *post_args(e2n1), ST, x1,
              params['W_cls'][:DIM], params['W_cls'][DIM:], row(params['b_cls']))
    return out


# speedup vs baseline: 2.3261x; 2.3261x over previous
import functools

import jax
import jax.numpy as jnp
import numpy as np
from jax import lax
from jax.experimental import pallas as pl
from jax.experimental.pallas import tpu as pltpu
from jax.experimental.pallas import tpu_sc as plsc

N_NODES = 10000
N_HEDGES = 5000
N_INC = 320000
FEAT = 128
DIM = 128
HEADS = 4
HID = DIM // HEADS
NCLS = 40
NLAYERS = 2

WID = DIM + 16
NW = 32
CHUNK = 80
ZROWS = 640

_S_np = np.zeros((DIM, HEADS), np.float32)
for _h in range(HEADS):
    _S_np[_h * HID:(_h + 1) * HID, _h] = 1.0


def _ln(o, g, b):
    m = jnp.mean(o, axis=-1, keepdims=True)
    c = o - m
    v = jnp.mean(c * c, axis=-1, keepdims=True)
    return c * jax.lax.rsqrt(v + 1e-5) * g + b


def _elu(o):
    return jnp.where(o > 0, o, jnp.exp(jnp.minimum(o, 0.0)) - 1.0)


def _pre_block(x, Wk, Wv, att, S, ST):
    Kx = jnp.dot(x, Wk, preferred_element_type=jnp.float32)
    alpha = jnp.dot(Kx * att, S, preferred_element_type=jnp.float32)
    gmax = jnp.max(alpha, axis=0, keepdims=True)
    ea = jnp.exp(alpha - gmax)
    Vx = jnp.dot(x, Wv, preferred_element_type=jnp.float32)
    eaexp = jnp.dot(ea, ST, preferred_element_type=jnp.float32)
    pad = jnp.zeros((x.shape[0], WID - DIM - HEADS), jnp.float32)
    return jnp.concatenate([Vx * eaexp, ea, pad], axis=1)


def _post_block(raw0, raw1, att, ST, ln0_g, ln0_b, Wff, bff, ln1_g, ln1_b):
    raw = raw0 + raw1
    s = jnp.dot(raw[:, DIM:DIM + HEADS], ST,
                preferred_element_type=jnp.float32) + 1e-16
    o = raw[:, :DIM] / s + att
    o = _ln(o, ln0_g, ln0_b)
    o = o + jax.nn.relu(jnp.dot(o, Wff, preferred_element_type=jnp.float32) + bff)
    o = _ln(o, ln1_g, ln1_b)
    return _elu(o)


def _t0_body(x_ref, bng, bnb, bnm, bnv, Wk, Wv, att, S, ST, u_ref):
    x = (x_ref[...] - bnm[...]) * jax.lax.rsqrt(bnv[...] + 1e-5) * bng[...] + bnb[...]
    u_ref[...] = _pre_block(x, Wk[...], Wv[...], att[...], S[...], ST[...])


def _tmid_body(r0, r1, att_a, g0, b0, Wff, bff, g1, b1,
               Wk, Wv, att_b, S, ST, u_ref, x_ref):
    x = _post_block(r0[...], r1[...], att_a[...], ST[...], g0[...], b0[...],
                    Wff[...], bff[...], g1[...], b1[...])
    x_ref[...] = x
    u_ref[...] = _pre_block(x, Wk[...], Wv[...], att_b[...], S[...], ST[...])


def _t4_body(r0, r1, att_a, g0, b0, Wff, bff, g1, b1, ST,
             x1_ref, Wc1, Wc2, bc, out_ref):
    x2 = _post_block(r0[...], r1[...], att_a[...], ST[...], g0[...], b0[...],
                     Wff[...], bff[...], g1[...], b1[...])
    logits = (jnp.dot(x1_ref[...], Wc1[...], preferred_element_type=jnp.float32)
              + jnp.dot(x2, Wc2[...], preferred_element_type=jnp.float32)
              + bc[...])
    z = logits - jnp.max(logits, axis=-1, keepdims=True)
    out_ref[...] = z - jnp.log(jnp.sum(jnp.exp(z), axis=-1, keepdims=True))


def _tc(body, out_shape, *args):
    return pl.pallas_call(body, out_shape=out_shape)(*args)


@functools.lru_cache(maxsize=None)
def _make_sc_segsum(nseg, n_rows):
    chunk = 128 if nseg <= 5000 else 64
    nseg_pad = ((nseg + 127) // 128) * 128
    rpt = nseg_pad // 16
    nsup = -(-N_INC // (NW * chunk))
    nsup += nsup % 2
    crows = nsup
    inc_pad = NW * crows * chunk

    mesh = plsc.VectorSubcoreMesh(core_axis_name="c", subcore_axis_name="s")

    @functools.partial(
        pl.kernel,
        mesh=mesh,
        compiler_params=pltpu.CompilerParams(use_tc_tiling_on_sc=False),
        out_type=jax.ShapeDtypeStruct((2 * nseg_pad, WID), jnp.float32),
        scratch_types=[
            pltpu.VMEM((crows, chunk), jnp.int32),
            pltpu.VMEM((crows, chunk), jnp.int32),
            [pltpu.VMEM((chunk, WID), jnp.float32)] * 2,
            pltpu.VMEM_SHARED((nseg_pad, WID), jnp.float32),
            [pltpu.SemaphoreType.DMA] * 2,
        ],
    )
    def segsum(u_hbm, src_hbm, dst_hbm, zeros_hbm, out_hbm,
               src_v, dst_v, bufs, acc, sems):
        c = lax.axis_index("c")
        s = lax.axis_index("s")
        w = c * 16 + s

        pltpu.sync_copy(src_hbm.at[pl.ds(w * crows, crows)], src_v)
        pltpu.sync_copy(dst_hbm.at[pl.ds(w * crows, crows)], dst_v)
        pltpu.sync_copy(zeros_hbm.at[pl.ds(0, rpt)], acc.at[pl.ds(s * rpt, rpt)])
        plsc.subcore_barrier()

        def fire_g(sup, bk):
            pltpu.async_copy(u_hbm.at[src_v.at[sup]], bufs[bk], sems[bk])

        def drain_g(sup, bk):
            pltpu.make_async_copy(u_hbm.at[src_v.at[sup]], bufs[bk],
                                  sems[bk]).wait()

        def scat(sup, bk):
            pltpu.sync_copy(bufs[bk], acc.at[dst_v.at[sup]], add=True)

        fire_g(0, 0)

        def body(t, carry):
            s1 = 2 * t - 1
            fire_g(s1, 1)
            drain_g(s1 - 1, 0)
            scat(s1 - 1, 0)
            fire_g(s1 + 1, 0)
            drain_g(s1, 1)
            scat(s1, 1)
            return carry

        lax.fori_loop(1, nsup // 2, body, 0)

        fire_g(nsup - 1, 1)
        drain_g(nsup - 2, 0)
        scat(nsup - 2, 0)
        drain_g(nsup - 1, 1)
        scat(nsup - 1, 1)
        plsc.subcore_barrier()

        row0 = c * nseg_pad + s * rpt
        pltpu.sync_copy(acc.at[pl.ds(s * rpt, rpt)], out_hbm.at[pl.ds(row0, rpt)])

    return segsum, nseg_pad, inc_pad, chunk


def _sc_segsum(U, src, dst, nseg, zeros):
    fn, nseg_pad, inc_pad, chunk = _make_sc_segsum(nseg, U.shape[0])
    pad = inc_pad - N_INC
    ar = jnp.arange(pad, dtype=dst.dtype)
    dump = nseg + ar % (nseg_pad - nseg)
    filler = ar % jnp.int32(U.shape[0])
    srcr = jnp.concatenate([src, filler]).reshape(-1, chunk)
    dstr = jnp.concatenate([dst, dump]).reshape(-1, chunk)
    out = fn(U, srcr, dstr, zeros)
    return out[:nseg], out[nseg_pad:nseg_pad + nseg]


def kernel(node_x, n2e_nodes_map, n2e_batch, e2n_edges_map, e2n_batch, params):
    S = jnp.asarray(_S_np)
    ST = jnp.asarray(_S_np.T)
    zeros = jnp.zeros((ZROWS, WID), jnp.float32)

    def row(v):
        return jnp.reshape(v, (1, -1)).astype(jnp.float32)

    def pre_args(p):
        return (p['Wk'], p['Wv'], row(p['att_r']), S, ST)

    def post_args(p):
        return (row(p['att_r']), row(p['ln0_g']), row(p['ln0_b']),
                p['Wff'], row(p['bff']), row(p['ln1_g']), row(p['ln1_b']))

    n2e0, n2e1 = params['n2e']
    e2n0, e2n1 = params['e2n']

    U_a = _tc(_t0_body, jax.ShapeDtypeStruct((N_NODES, WID), jnp.float32),
              node_x, row(params['bn_g']), row(params['bn_b']),
              row(params['bn_m']), row(params['bn_v']), *pre_args(n2e0))
    ra0, ra1 = _sc_segsum(U_a, n2e_nodes_map, n2e_batch, N_HEDGES, zeros)

    U_b, _ = _tc(_tmid_body,
                 (jax.ShapeDtypeStruct((N_HEDGES, WID), jnp.float32),
                  jax.ShapeDtypeStruct((N_HEDGES, DIM), jnp.float32)),
                 ra0, ra1, *post_args(n2e0), *pre_args(e2n0))
    rb0, rb1 = _sc_segsum(U_b, e2n_edges_map, e2n_batch, N_NODES, zeros)

    U_c, x1 = _tc(_tmid_body,
                  (jax.ShapeDtypeStruct((N_NODES, WID), jnp.float32),
                   jax.ShapeDtypeStruct((N_NODES, DIM), jnp.float32)),
                  rb0, rb1, *post_args(e2n0), *pre_args(n2e1))
    rc0, rc1 = _sc_segsum(U_c, n2e_nodes_map, n2e_batch, N_HEDGES, zeros)

    U_d, _ = _tc(_tmid_body,
                 (jax.ShapeDtypeStruct((N_HEDGES, WID), jnp.float32),
                  jax.ShapeDtypeStruct((N_HEDGES, DIM), jnp.float32)),
                 rc0, rc1, *post_args(n2e1), *pre_args(e2n1))
    rd0, rd1 = _sc_segsum(U_d, e2n_edges_map, e2n_batch, N_NODES, zeros)

    out = _tc(_t4_body, jax.ShapeDtypeStruct((N_NODES, NCLS), jnp.float32),
              rd0, rd1, *post_args(e2n1), ST, x1,
              params['W_cls'][:DIM], params['W_cls'][DIM:], row(params['b_cls']))
    return out
